# Initial kernel scaffold; baseline (speedup 1.0000x reference)
import jax
import jax.numpy as jnp
from jax.experimental import pallas as pl


def kernel(features, ei_p_citing, ei_p_pa, ei_a_ap, W0_citing, W0_pa, W0_ap, W1_citing, W1_pa, W1_ap, Wfc_p, bfc_p, Wfc_a, bfc_a):
    raise NotImplementedError("write your pallas kernel here")


# speedup vs baseline: 6.3650x; 6.3650x over previous
import functools

import jax
import jax.numpy as jnp
from jax import lax
from jax.experimental import pallas as pl
from jax.experimental.pallas import tpu as pltpu
from jax.experimental.pallas import tpu_sc as plsc

N = 25000
E = 400000
FT = 128
HALF = 64
OUT = 128

NPAD = 25088
RPT = NPAD // 16
KSUB = 125
NSUB = 8
OUTER = 25
IDXROWS = E // KSUB
IDXRPT = IDXROWS // 16
ZROWS = 98

BM = 200
GRID = N // BM


def _make_spmm(with_cnt):
    mesh = plsc.VectorSubcoreMesh(core_axis_name="c", subcore_axis_name="s")
    out_type = [jax.ShapeDtypeStruct((NPAD, HALF), jnp.float32),
                jax.ShapeDtypeStruct((NPAD, HALF), jnp.float32)]
    if with_cnt:
        out_type.append(jax.ShapeDtypeStruct((NPAD,), jnp.float32))
    scratch_types = [
        pltpu.VMEM((NSUB, KSUB), jnp.int32),
        pltpu.VMEM((NSUB, KSUB), jnp.int32),
        pltpu.VMEM((2, KSUB, HALF), jnp.float32),
        pltpu.VMEM((ZROWS, HALF), jnp.float32),
        pltpu.VMEM((RPT,), jnp.float32),
        pltpu.VMEM((128,), jnp.float32),
        pltpu.VMEM_SHARED((NPAD, HALF), jnp.float32),
        pltpu.VMEM_SHARED((NPAD,), jnp.float32),
        pltpu.SemaphoreType.DMA,
        pltpu.SemaphoreType.DMA,
    ]

    def body(row_hbm, col_hbm, tlo, thi, *rest):
        if with_cnt:
            out_lo, out_hi, out_cnt = rest[:3]
            rest = rest[3:]
        else:
            out_lo, out_hi = rest[:2]
            rest = rest[2:]
        rowv, colv, gbuf, zbuf, zc, ones, acc, acc_cnt, sem0, sem1 = rest
        sems = (sem0, sem1)
        c = lax.axis_index("c")
        s = lax.axis_index("s")

        def zrow(r, carry):
            for j in range(HALF // 16):
                zbuf[r, pl.ds(j * 16, 16)] = jnp.zeros((16,), jnp.float32)
            return carry
        lax.fori_loop(0, ZROWS, zrow, 0)

        for b in range(RPT // ZROWS):
            pltpu.sync_copy(zbuf, acc.at[pl.ds(s * RPT + b * ZROWS, ZROWS)])

        if with_cnt:
            def zfill(i, carry):
                zc[pl.ds(i * 16, 16)] = jnp.zeros((16,), jnp.float32)
                return carry
            lax.fori_loop(0, RPT // 16, zfill, 0)
            for j in range(128 // 16):
                ones[pl.ds(j * 16, 16)] = jnp.ones((16,), jnp.float32)
            pltpu.sync_copy(zc, acc_cnt.at[pl.ds(s * RPT, RPT)])

        plsc.subcore_barrier()

        def accumulate(table, do_cnt):
            def outer(i, carry):
                r0 = s * IDXRPT + i * NSUB
                pltpu.sync_copy(row_hbm.at[pl.ds(r0, NSUB)], rowv)
                pltpu.sync_copy(col_hbm.at[pl.ds(r0, NSUB)], colv)
                cd = pltpu.async_copy(table.at[colv.at[0]], gbuf.at[0],
                                      sems[0])
                for j in range(NSUB):
                    if j + 1 < NSUB:
                        nxt = pltpu.async_copy(table.at[colv.at[j + 1]],
                                               gbuf.at[(j + 1) % 2],
                                               sems[(j + 1) % 2])
                    cd.wait()
                    pltpu.sync_copy(gbuf.at[j % 2], acc.at[rowv.at[j]],
                                    add=True)
                    if do_cnt:
                        pltpu.sync_copy(ones.at[pl.ds(0, KSUB)],
                                        acc_cnt.at[rowv.at[j]], add=True)
                    if j + 1 < NSUB:
                        cd = nxt
                return carry
            lax.fori_loop(0, OUTER, outer, 0)

        @pl.when(c == 0)
        def _():
            accumulate(tlo, with_cnt)

        @pl.when(c == 1)
        def _():
            accumulate(thi, False)

        plsc.subcore_barrier()

        @pl.when(c == 0)
        def _():
            pltpu.sync_copy(acc.at[pl.ds(s * RPT, RPT)],
                            out_lo.at[pl.ds(s * RPT, RPT)])
            if with_cnt:
                pltpu.sync_copy(acc_cnt.at[pl.ds(s * RPT, RPT)],
                                out_cnt.at[pl.ds(s * RPT, RPT)])

        @pl.when(c == 1)
        def _():
            pltpu.sync_copy(acc.at[pl.ds(s * RPT, RPT)],
                            out_hi.at[pl.ds(s * RPT, RPT)])

    return functools.partial(
        pl.kernel, mesh=mesh, out_type=out_type,
        scratch_types=scratch_types,
        compiler_params=pltpu.CompilerParams(use_tc_tiling_on_sc=False),
    )(body)


_SPMM_CNT = _make_spmm(True)
_SPMM = _make_spmm(False)


def _lrelu(x):
    return jnp.where(x >= 0, x, 0.01 * x)


def _mm_split(x, w):
    def body(x_ref, w_ref, lo_ref, hi_ref):
        y = jnp.dot(x_ref[...], w_ref[...], preferred_element_type=jnp.float32)
        lo_ref[...] = y[:, :HALF]
        hi_ref[...] = y[:, HALF:]
    return pl.pallas_call(
        body,
        grid=(GRID,),
        in_specs=[pl.BlockSpec((BM, FT), lambda i: (i, 0)),
                  pl.BlockSpec((FT, FT), lambda i: (0, 0))],
        out_specs=[pl.BlockSpec((BM, HALF), lambda i: (i, 0)),
                   pl.BlockSpec((BM, HALF), lambda i: (i, 0))],
        out_shape=[jax.ShapeDtypeStruct((N, HALF), jnp.float32)] * 2,
    )(x, w)


def _mid_p(slo_c, shi_c, cnt_c, slo_pa, shi_pa, cnt_pa, w1, w3):
    def body(slc, shc, cc, slp, shp, cp, w1_ref, w3_ref, o1l, o1h, o3l, o3h):
        rc = 1.0 / jnp.maximum(cc[...], 1.0)
        rp = 1.0 / jnp.maximum(cp[...], 1.0)
        el = (_lrelu(slc[...] * rc) + _lrelu(slp[...] * rp)) * 0.5
        eh = (_lrelu(shc[...] * rc) + _lrelu(shp[...] * rp)) * 0.5
        e = jnp.concatenate([el, eh], axis=1)
        t1 = jnp.dot(e, w1_ref[...], preferred_element_type=jnp.float32)
        t3 = jnp.dot(e, w3_ref[...], preferred_element_type=jnp.float32)
        o1l[...] = t1[:, :HALF]
        o1h[...] = t1[:, HALF:]
        o3l[...] = t3[:, :HALF]
        o3h[...] = t3[:, HALF:]
    half_in = pl.BlockSpec((BM, HALF), lambda i: (i, 0))
    cnt_in = pl.BlockSpec((BM, 1), lambda i: (i, 0))
    w_in = pl.BlockSpec((FT, FT), lambda i: (0, 0))
    half_out = pl.BlockSpec((BM, HALF), lambda i: (i, 0))
    return pl.pallas_call(
        body,
        grid=(GRID,),
        in_specs=[half_in, half_in, cnt_in, half_in, half_in, cnt_in,
                  w_in, w_in],
        out_specs=[half_out] * 4,
        out_shape=[jax.ShapeDtypeStruct((N, HALF), jnp.float32)] * 4,
    )(slo_c, shi_c, cnt_c, slo_pa, shi_pa, cnt_pa, w1, w3)


def _mid_a(slo, shi, cnt, w):
    def body(sl, sh, cc, w_ref, ol, oh):
        r = 1.0 / jnp.maximum(cc[...], 1.0)
        e = jnp.concatenate([_lrelu(sl[...] * r), _lrelu(sh[...] * r)], axis=1)
        t = jnp.dot(e, w_ref[...], preferred_element_type=jnp.float32)
        ol[...] = t[:, :HALF]
        oh[...] = t[:, HALF:]
    half_in = pl.BlockSpec((BM, HALF), lambda i: (i, 0))
    cnt_in = pl.BlockSpec((BM, 1), lambda i: (i, 0))
    w_in = pl.BlockSpec((FT, FT), lambda i: (0, 0))
    return pl.pallas_call(
        body,
        grid=(GRID,),
        in_specs=[half_in, half_in, cnt_in, w_in],
        out_specs=[pl.BlockSpec((BM, HALF), lambda i: (i, 0))] * 2,
        out_shape=[jax.ShapeDtypeStruct((N, HALF), jnp.float32)] * 2,
    )(slo, shi, cnt, w)


def _fin_two(zlo_a, zhi_a, cnt_a, zlo_b, zhi_b, cnt_b, feat, wfc, bias):
    def body(zla, zha, ca, zlb, zhb, cb, f_ref, w_ref, b_ref, o_ref):
        ra = 1.0 / jnp.maximum(ca[...], 1.0)
        rb = 1.0 / jnp.maximum(cb[...], 1.0)
        vl = (_lrelu(zla[...] * ra) + _lrelu(zlb[...] * rb)) * 0.5
        vh = (_lrelu(zha[...] * ra) + _lrelu(zhb[...] * rb)) * 0.5
        v = jnp.concatenate([vl, vh], axis=1)
        w = w_ref[...]
        o_ref[...] = (jnp.dot(v, w[:FT], preferred_element_type=jnp.float32)
                      + jnp.dot(f_ref[...], w[FT:],
                                preferred_element_type=jnp.float32)
                      + b_ref[...])
    half_in = pl.BlockSpec((BM, HALF), lambda i: (i, 0))
    cnt_in = pl.BlockSpec((BM, 1), lambda i: (i, 0))
    return pl.pallas_call(
        body,
        grid=(GRID,),
        in_specs=[half_in, half_in, cnt_in, half_in, half_in, cnt_in,
                  pl.BlockSpec((BM, FT), lambda i: (i, 0)),
                  pl.BlockSpec((FT + FT, OUT), lambda i: (0, 0)),
                  pl.BlockSpec((1, OUT), lambda i: (0, 0))],
        out_specs=pl.BlockSpec((BM, OUT), lambda i: (i, 0)),
        out_shape=jax.ShapeDtypeStruct((N, OUT), jnp.float32),
    )(zlo_a, zhi_a, cnt_a, zlo_b, zhi_b, cnt_b, feat, wfc, bias)


def _fin_one(zlo, zhi, cnt, feat, wfc, bias):
    def body(zl, zh, cc, f_ref, w_ref, b_ref, o_ref):
        r = 1.0 / jnp.maximum(cc[...], 1.0)
        v = jnp.concatenate([_lrelu(zl[...] * r), _lrelu(zh[...] * r)], axis=1)
        w = w_ref[...]
        o_ref[...] = (jnp.dot(v, w[:FT], preferred_element_type=jnp.float32)
                      + jnp.dot(f_ref[...], w[FT:],
                                preferred_element_type=jnp.float32)
                      + b_ref[...])
    return pl.pallas_call(
        body,
        grid=(GRID,),
        in_specs=[pl.BlockSpec((BM, HALF), lambda i: (i, 0)),
                  pl.BlockSpec((BM, HALF), lambda i: (i, 0)),
                  pl.BlockSpec((BM, 1), lambda i: (i, 0)),
                  pl.BlockSpec((BM, FT), lambda i: (i, 0)),
                  pl.BlockSpec((FT + FT, OUT), lambda i: (0, 0)),
                  pl.BlockSpec((1, OUT), lambda i: (0, 0))],
        out_specs=pl.BlockSpec((BM, OUT), lambda i: (i, 0)),
        out_shape=jax.ShapeDtypeStruct((N, OUT), jnp.float32),
    )(zlo, zhi, cnt, feat, wfc, bias)


def kernel(features, ei_p_citing, ei_p_pa, ei_a_ap,
           W0_citing, W0_pa, W0_ap,
           W1_citing, W1_pa, W1_ap,
           Wfc_p, bfc_p, Wfc_a, bfc_a):
    feat_p = features[:N]
    feat_a = features[N:]
    r_c = ei_p_citing[0].reshape(IDXROWS, KSUB)
    c_c = ei_p_citing[1].reshape(IDXROWS, KSUB)
    r_pa = ei_p_pa[0].reshape(IDXROWS, KSUB)
    c_pa = ei_p_pa[1].reshape(IDXROWS, KSUB)
    r_ap = ei_a_ap[0].reshape(IDXROWS, KSUB)
    c_ap = ei_a_ap[1].reshape(IDXROWS, KSUB)

    y_c = _mm_split(feat_p, W0_citing)
    y_pa = _mm_split(feat_a, W0_pa)
    y_ap = _mm_split(feat_p, W0_ap)
    s_c_lo, s_c_hi, cnt_c = _SPMM_CNT(r_c, c_c, *y_c)
    s_pa_lo, s_pa_hi, cnt_pa = _SPMM_CNT(r_pa, c_pa, *y_pa)
    s_ap_lo, s_ap_hi, cnt_ap = _SPMM_CNT(r_ap, c_ap, *y_ap)
    cc2 = cnt_c.reshape(NPAD, 1)
    cpa2 = cnt_pa.reshape(NPAD, 1)
    cap2 = cnt_ap.reshape(NPAD, 1)

    t1_lo, t1_hi, t3_lo, t3_hi = _mid_p(
        s_c_lo, s_c_hi, cc2, s_pa_lo, s_pa_hi, cpa2, W1_citing, W1_ap)
    t2_lo, t2_hi = _mid_a(s_ap_lo, s_ap_hi, cap2, W1_pa)

    z1c_lo, z1c_hi = _SPMM(r_c, c_c, t1_lo, t1_hi)
    z1pa_lo, z1pa_hi = _SPMM(r_pa, c_pa, t2_lo, t2_hi)
    z1ap_lo, z1ap_hi = _SPMM(r_ap, c_ap, t3_lo, t3_hi)

    out_p = _fin_two(z1c_lo, z1c_hi, cc2, z1pa_lo, z1pa_hi, cpa2,
                     feat_p, Wfc_p, bfc_p.reshape(1, OUT))
    out_a = _fin_one(z1ap_lo, z1ap_hi, cap2, feat_a, Wfc_a,
                     bfc_a.reshape(1, OUT))
    return jnp.concatenate([out_p, out_a], axis=0)
